# baseline (speedup 1.0000x reference)
import functools

import jax
import jax.numpy as jnp
from jax import lax
from jax.experimental import pallas as pl
from jax.experimental.pallas import tpu as pltpu
from jax.experimental.pallas import tpu_sc as plsc

N_TABLES = 4
VOCAB = 100000
DIM = 64

NC = 2
NS = 16
NW = NC * NS
L = 16

D_PER_W = DIM * N_TABLES // NW
STAGE = 4096
N_STAGES = 4


def _grouped_gather_t(values, tab_t):
    b = values.shape[0]
    bt = b // N_TABLES
    mesh = plsc.VectorSubcoreMesh(core_axis_name="c", subcore_axis_name="s")

    @functools.partial(
        pl.kernel,
        mesh=mesh,
        compiler_params=pltpu.CompilerParams(needs_layout_passes=False),
        out_type=jax.ShapeDtypeStruct((DIM, b), jnp.float32),
        scratch_types=[
            pltpu.VMEM((bt,), jnp.int32),
            pltpu.VMEM((VOCAB,), jnp.float32),
            pltpu.VMEM((2, STAGE), jnp.float32),
            pltpu.SemaphoreType.DMA,
            pltpu.SemaphoreType.DMA,
        ],
    )
    def k(vals_hbm, tab_hbm, out_hbm, idx_v, row_v, stage_v, sem_in, sem_out):
        wid = lax.axis_index("s") * NC + lax.axis_index("c")
        t = wid // D_PER_W
        db = wid % D_PER_W
        r0 = t * DIM + db * D_PER_W
        pltpu.sync_copy(vals_hbm.at[pl.ds(t * bt, bt)], idx_v)
        pltpu.async_copy(tab_hbm.at[r0], row_v, sem_in)

        def do_dim(j, carry):
            d = db * D_PER_W + j
            r = t * DIM + d
            pltpu.make_async_copy(tab_hbm.at[r], row_v, sem_in).wait()
            for s in range(N_STAGES):
                if s >= 2:
                    pltpu.make_async_copy(
                        stage_v.at[s % 2],
                        out_hbm.at[d, pl.ds(t * bt, STAGE)],
                        sem_out,
                    ).wait()
                @plsc.parallel_loop(0, STAGE // L, unroll=8)
                def _(g):
                    iv = idx_v[pl.ds(s * STAGE + g * L, L)]
                    stage_v[s % 2, pl.ds(g * L, L)] = plsc.load_gather(
                        row_v, [iv])
                pltpu.async_copy(
                    stage_v.at[s % 2],
                    out_hbm.at[d, pl.ds(t * bt + s * STAGE, STAGE)],
                    sem_out,
                )
            @pl.when(j + 1 < D_PER_W)
            def _():
                pltpu.async_copy(tab_hbm.at[r + 1], row_v, sem_in)
            for _ in range(2):
                pltpu.make_async_copy(
                    stage_v.at[0],
                    out_hbm.at[d, pl.ds(t * bt, STAGE)],
                    sem_out,
                ).wait()
            return carry

        lax.fori_loop(0, D_PER_W, do_dim, 0)

    return k(values, tab_t)


def kernel(values, tables):
    tab_t = jnp.transpose(tables, (0, 2, 1)).reshape(N_TABLES * DIM, VOCAB)
    out_t = _grouped_gather_t(values, tab_t)
    return jnp.transpose(out_t)


# speedup vs baseline: 1.0164x; 1.0164x over previous
import functools

import jax
import jax.numpy as jnp
from jax import lax
from jax.experimental import pallas as pl
from jax.experimental.pallas import tpu as pltpu
from jax.experimental.pallas import tpu_sc as plsc

N_TABLES = 4
VOCAB = 100000
DIM = 64

NC = 2
NS = 16
NW = NC * NS
L = 16

D_PER_W = DIM * N_TABLES // NW
STAGE = 4096
N_STAGES = 4


def _grouped_gather_t(values, tab_t):
    b = values.shape[0]
    bt = b // N_TABLES
    mesh = plsc.VectorSubcoreMesh(core_axis_name="c", subcore_axis_name="s")

    @functools.partial(
        pl.kernel,
        mesh=mesh,
        compiler_params=pltpu.CompilerParams(needs_layout_passes=False),
        out_type=jax.ShapeDtypeStruct((DIM, b), jnp.float32),
        scratch_types=[
            pltpu.VMEM((bt,), jnp.int32),
            pltpu.VMEM((VOCAB,), jnp.float32),
            pltpu.VMEM((2, STAGE), jnp.float32),
            pltpu.SemaphoreType.DMA,
            pltpu.SemaphoreType.DMA,
        ],
    )
    def k(vals_hbm, tab_hbm, out_hbm, idx_v, row_v, stage_v, sem_in, sem_out):
        wid = lax.axis_index("s") * NC + lax.axis_index("c")
        t = wid // D_PER_W
        db = wid % D_PER_W
        r0 = t * DIM + db * D_PER_W
        pltpu.async_copy(tab_hbm.at[r0], row_v, sem_in)
        pltpu.sync_copy(vals_hbm.at[pl.ds(t * bt, bt)], idx_v)

        def do_dim(j, carry):
            d = db * D_PER_W + j
            r = t * DIM + d
            pltpu.make_async_copy(tab_hbm.at[r], row_v, sem_in).wait()
            for s in range(N_STAGES):
                if s >= 2:
                    pltpu.make_async_copy(
                        stage_v.at[s % 2],
                        out_hbm.at[d, pl.ds(t * bt, STAGE)],
                        sem_out,
                    ).wait()
                @plsc.parallel_loop(0, STAGE // L, unroll=16)
                def _(g):
                    iv = idx_v[pl.ds(s * STAGE + g * L, L)]
                    stage_v[s % 2, pl.ds(g * L, L)] = plsc.load_gather(
                        row_v, [iv])
                pltpu.async_copy(
                    stage_v.at[s % 2],
                    out_hbm.at[d, pl.ds(t * bt + s * STAGE, STAGE)],
                    sem_out,
                )
            @pl.when(j + 1 < D_PER_W)
            def _():
                pltpu.async_copy(tab_hbm.at[r + 1], row_v, sem_in)
            for _ in range(2):
                pltpu.make_async_copy(
                    stage_v.at[0],
                    out_hbm.at[d, pl.ds(t * bt, STAGE)],
                    sem_out,
                ).wait()
            return carry

        lax.fori_loop(0, D_PER_W, do_dim, 0)

    return k(values, tab_t)


def kernel(values, tables):
    tab_t = jnp.transpose(tables, (0, 2, 1)).reshape(N_TABLES * DIM, VOCAB)
    out_t = _grouped_gather_t(values, tab_t)
    return jnp.transpose(out_t)
